# baseline (speedup 1.0000x reference)
import functools

import jax
import jax.numpy as jnp
from jax import lax
from jax.experimental import pallas as pl
from jax.experimental.pallas import tpu as pltpu
from jax.experimental.pallas import tpu_sc as plsc

M_EDGES = 5000
NS = 16
K = 80
ROWS = 8


def _strided_chunks(sid, total_chunks, body):
    per = (total_chunks + NS - 1) // NS

    def loop_body(i, carry):
        ch = sid + i * NS

        @pl.when(ch < total_chunks)
        def _():
            body(ch)

        return carry

    lax.fori_loop(0, per, loop_body, 0)


@functools.lru_cache(maxsize=None)
def _make_degree_kernel(N, M, NNZ):
    PT = NNZ // NS
    PC = PT // K
    assert NNZ % NS == 0 and PT % K == 0 and PC % 8 == 5
    mesh = plsc.VectorSubcoreMesh(core_axis_name="c", subcore_axis_name="s")

    @functools.partial(
        pl.kernel,
        out_type=(jax.ShapeDtypeStruct((N, 128), jnp.float32),
                  jax.ShapeDtypeStruct((M, 128), jnp.float32)),
        mesh=mesh,
        scratch_types=(
            [pltpu.VMEM((K,), jnp.int32)] * 8
            + [pltpu.VMEM((ROWS, 128), jnp.float32),
               pltpu.VMEM((K, 128), jnp.float32),
               pltpu.VMEM_SHARED((N, 128), jnp.float32)]
            + [pltpu.SemaphoreType.DMA] * 12
        ),
    )
    def deg_kernel(ids2, dv16, de16,
                   x0, x1, x2, x3, x4, x5, x6, x7, zb, ones, hist,
                   s0, s1, s2, s3, j0, j1, j2, j3, j4, j5, j6, j7):
        c = lax.axis_index("c")
        s = lax.axis_index("s")
        sxs = (x0, x1, x2, x3, x4, x5, x6, x7)
        ssems = (s0, s1, s2, s3)
        jsems = (j0, j1, j2, j3, j4, j5, j6, j7)
        base = c * NNZ + s * PT
        for r in range(ROWS):
            for cc in range(8):
                zb[r, pl.ds(cc * 16, 16)] = jnp.zeros((16,), jnp.float32)
        for r in range(K):
            for cc in range(8):
                ones[r, pl.ds(cc * 16, 16)] = jnp.ones((16,), jnp.float32)

        def zero_chunk(ch):
            pltpu.sync_copy(zb, hist.at[pl.ds(ch * ROWS, ROWS)])

        _strided_chunks(s, N // ROWS, zero_chunk)
        plsc.subcore_barrier()

        def load_idx(i, b8):
            pltpu.async_copy(ids2.at[pl.ds(base + i * K, K)], sxs[b8],
                             jsems[b8])

        def wait_idx(i, b8):
            pltpu.make_async_copy(ids2.at[pl.ds(base + i * K, K)],
                                  sxs[b8], jsems[b8]).wait()

        def chunk(i, b8):
            b4 = b8 % 4
            nb = (b8 + 4) % 8

            @pl.when(i >= 4)
            def _():
                pltpu.make_async_copy(ones, hist.at[sxs[nb]],
                                      ssems[b4]).wait()

            @pl.when(i + 4 < PC)
            def _():
                load_idx(i + 4, nb)

            wait_idx(i, b8)
            pltpu.async_copy(ones, hist.at[sxs[b8]], ssems[b4], add=True)

        for b in range(4):
            load_idx(b, b)

        def octet(o, carry):
            for b8 in range(8):
                chunk(8 * o + b8, b8)
            return carry

        lax.fori_loop(0, PC // 8, octet, 0)
        for i in range(PC - PC % 8, PC):
            chunk(i, i % 8)
        for i in range(PC - 4, PC):
            pltpu.make_async_copy(ones, hist.at[sxs[i % 8]],
                                  ssems[i % 4]).wait()
        plsc.subcore_barrier()

        @pl.when(c == 0)
        def _():
            def out_v(ch):
                pltpu.sync_copy(hist.at[pl.ds(ch * ROWS, ROWS)],
                                dv16.at[pl.ds(ch * ROWS, ROWS)])

            _strided_chunks(s, N // ROWS, out_v)

        @pl.when(c == 1)
        def _():
            def out_e(ch):
                pltpu.sync_copy(hist.at[pl.ds(ch * ROWS, ROWS)],
                                de16.at[pl.ds(ch * ROWS, ROWS)])

            _strided_chunks(s, M // ROWS, out_e)

    return deg_kernel


@functools.lru_cache(maxsize=None)
def _make_smooth_kernel(N, M, NNZ, Dh):
    PT = NNZ // NS
    PC = PT // K
    assert NNZ % NS == 0 and PT % K == 0 and PC % 8 == 5
    assert Dh % 16 == 0
    mesh = plsc.VectorSubcoreMesh(core_axis_name="c", subcore_axis_name="s")

    @functools.partial(
        pl.kernel,
        out_type=(jax.ShapeDtypeStruct((2 * N, Dh), jnp.float32),
                  jax.ShapeDtypeStruct((2 * M, Dh), jnp.float32)),
        mesh=mesh,
        scratch_types=(
            [pltpu.VMEM((K,), jnp.int32)] * 16
            + [pltpu.VMEM((K, Dh), jnp.float32)] * 4
            + [pltpu.VMEM((ROWS, Dh), jnp.float32),
               pltpu.VMEM((ROWS, 16), jnp.float32),
               pltpu.VMEM_SHARED((N, Dh), jnp.float32)]
            + [pltpu.SemaphoreType.DMA] * 24
        ),
    )
    def smooth(xh2, vg1, es1, eg1, vs1, de16, y2, eagg2, *bufs):
        gxs = bufs[0:8]
        sxs = bufs[8:16]
        rows = bufs[16:20]
        ebuf, dbuf, acc = bufs[20:23]
        gsems = bufs[23:27]
        ssems = bufs[27:31]
        isems = bufs[31:39]
        jsems = bufs[39:47]
        c = lax.axis_index("c")
        s = lax.axis_index("s")
        gbase = (c * NS + s) * PT
        sbase = s * PT

        for r in range(ROWS):
            for cc in range(Dh // 16):
                ebuf[r, pl.ds(cc * 16, 16)] = jnp.zeros((16,), jnp.float32)

        def zero_acc(ch):
            pltpu.sync_copy(ebuf, acc.at[pl.ds(ch * ROWS, ROWS)])

        _strided_chunks(s, M // ROWS, zero_acc)
        plsc.subcore_barrier()

        def pipe_phase(src, g1, s1):
            def load_gidx(i, b):
                pltpu.async_copy(g1.at[pl.ds(gbase + i * K, K)], gxs[b],
                                 isems[b])

            def wait_gidx(i, b):
                pltpu.make_async_copy(g1.at[pl.ds(gbase + i * K, K)],
                                      gxs[b], isems[b]).wait()

            def load_sidx(i, b):
                pltpu.async_copy(s1.at[pl.ds(sbase + i * K, K)], sxs[b],
                                 jsems[b])

            def wait_sidx(i, b):
                pltpu.make_async_copy(s1.at[pl.ds(sbase + i * K, K)],
                                      sxs[b], jsems[b]).wait()

            def chunk(i, b8):
                b4 = b8 % 4
                g2 = (b4 + 2) % 4

                @pl.when(i >= 2)
                def _():
                    pltpu.make_async_copy(rows[g2],
                                          acc.at[sxs[(b8 + 6) % 8]],
                                          ssems[g2]).wait()

                @pl.when(i + 6 < PC)
                def _():
                    load_sidx(i + 6, (b8 + 6) % 8)

                @pl.when(i + 2 < PC)
                def _():
                    wait_gidx(i + 2, (b8 + 2) % 8)
                    pltpu.async_copy(src.at[gxs[(b8 + 2) % 8]], rows[g2],
                                     gsems[g2])

                pltpu.make_async_copy(src.at[gxs[b8]], rows[b4],
                                      gsems[b4]).wait()

                @pl.when(i + 8 < PC)
                def _():
                    load_gidx(i + 8, b8)

                wait_sidx(i, b8)
                pltpu.async_copy(rows[b4], acc.at[sxs[b8]], ssems[b4],
                                 add=True)

            for b in range(8):
                load_gidx(b, b)
                load_sidx(b, b)
            wait_gidx(0, 0)
            pltpu.async_copy(src.at[gxs[0]], rows[0], gsems[0])
            wait_gidx(1, 1)
            pltpu.async_copy(src.at[gxs[1]], rows[1], gsems[1])

            def octet(o, carry):
                for b8 in range(8):
                    chunk(8 * o + b8, b8)
                return carry

            lax.fori_loop(0, PC // 8, octet, 0)
            for i in range(PC - PC % 8, PC):
                chunk(i, i % 8)
            for i in range(PC - 2, PC):
                pltpu.make_async_copy(rows[i % 4], acc.at[sxs[i % 8]],
                                      ssems[i % 4]).wait()

        eoff = c * M
        pipe_phase(xh2, vg1, es1)
        plsc.subcore_barrier()

        def scale_chunk(ch):
            r0 = ch * ROWS
            pltpu.sync_copy(acc.at[pl.ds(r0, ROWS)], ebuf)
            pltpu.sync_copy(de16.at[pl.ds(r0, ROWS)], dbuf)
            for r in range(ROWS):
                d = dbuf[r, pl.ds(0, 16)]
                dinv = jnp.where(d > 0.0, 1.0 / d, 0.0)
                for cc in range(Dh // 16):
                    ebuf[r, pl.ds(cc * 16, 16)] = (
                        ebuf[r, pl.ds(cc * 16, 16)] * dinv)
            pltpu.sync_copy(ebuf, eagg2.at[pl.ds(eoff + r0, ROWS)])

        _strided_chunks(s, M // ROWS, scale_chunk)
        plsc.subcore_barrier()

        for r in range(ROWS):
            for cc in range(Dh // 16):
                ebuf[r, pl.ds(cc * 16, 16)] = jnp.zeros((16,), jnp.float32)

        def zero_acc2(ch):
            pltpu.sync_copy(ebuf, acc.at[pl.ds(ch * ROWS, ROWS)])

        _strided_chunks(s, N // ROWS, zero_acc2)
        plsc.subcore_barrier()

        pipe_phase(eagg2, eg1, vs1)
        plsc.subcore_barrier()

        coff = c * N

        def write_out(ch):
            r0 = ch * ROWS
            pltpu.sync_copy(acc.at[pl.ds(r0, ROWS)],
                            y2.at[pl.ds(coff + r0, ROWS)])

        _strided_chunks(s, N // ROWS, write_out)

    return smooth


@functools.lru_cache(maxsize=None)
def _make_partial_agg(NNZ, ACC_R):
    K2 = 40
    PT2 = NNZ // (2 * NS)
    PC = PT2 // K2
    assert NNZ % (2 * NS) == 0 and PT2 % K2 == 0 and PC % 8 == 5
    mesh = plsc.VectorSubcoreMesh(core_axis_name="c", subcore_axis_name="s")

    @functools.partial(
        pl.kernel,
        out_type=jax.ShapeDtypeStruct((2 * ACC_R, 128), jnp.float32),
        mesh=mesh,
        scratch_types=(
            [pltpu.VMEM((K2,), jnp.int32)] * 16
            + [pltpu.VMEM((K2, 128), jnp.float32)] * 4
            + [pltpu.VMEM((ROWS, 128), jnp.float32),
               pltpu.VMEM_SHARED((ACC_R, 128), jnp.float32)]
            + [pltpu.SemaphoreType.DMA] * 24
        ),
    )
    def pagg(src, g1, s1, out, *bufs):
        gxs = bufs[0:8]
        sxs = bufs[8:16]
        rows = bufs[16:20]
        zbuf, acc = bufs[20:22]
        gsems = bufs[22:26]
        ssems = bufs[26:30]
        isems = bufs[30:38]
        jsems = bufs[38:46]
        c = lax.axis_index("c")
        s = lax.axis_index("s")
        base = (c * NS + s) * PT2

        for r in range(ROWS):
            for cc in range(8):
                zbuf[r, pl.ds(cc * 16, 16)] = jnp.zeros((16,), jnp.float32)

        def zero_acc(ch):
            pltpu.sync_copy(zbuf, acc.at[pl.ds(ch * ROWS, ROWS)])

        _strided_chunks(s, ACC_R // ROWS, zero_acc)
        plsc.subcore_barrier()

        def load_gidx(i, b):
            pltpu.async_copy(g1.at[pl.ds(base + i * K2, K2)], gxs[b],
                             isems[b])

        def wait_gidx(i, b):
            pltpu.make_async_copy(g1.at[pl.ds(base + i * K2, K2)],
                                  gxs[b], isems[b]).wait()

        def load_sidx(i, b):
            pltpu.async_copy(s1.at[pl.ds(base + i * K2, K2)], sxs[b],
                             jsems[b])

        def wait_sidx(i, b):
            pltpu.make_async_copy(s1.at[pl.ds(base + i * K2, K2)],
                                  sxs[b], jsems[b]).wait()

        def chunk(i, b8):
            b4 = b8 % 4
            g2 = (b4 + 2) % 4

            @pl.when(i >= 2)
            def _():
                pltpu.make_async_copy(rows[g2],
                                      acc.at[sxs[(b8 + 6) % 8]],
                                      ssems[g2]).wait()

            @pl.when(i + 6 < PC)
            def _():
                load_sidx(i + 6, (b8 + 6) % 8)

            @pl.when(i + 2 < PC)
            def _():
                wait_gidx(i + 2, (b8 + 2) % 8)
                pltpu.async_copy(src.at[gxs[(b8 + 2) % 8]], rows[g2],
                                 gsems[g2])

            pltpu.make_async_copy(src.at[gxs[b8]], rows[b4],
                                  gsems[b4]).wait()

            @pl.when(i + 8 < PC)
            def _():
                load_gidx(i + 8, b8)

            wait_sidx(i, b8)
            pltpu.async_copy(rows[b4], acc.at[sxs[b8]], ssems[b4],
                             add=True)

        for b in range(8):
            load_gidx(b, b)
            load_sidx(b, b)
        wait_gidx(0, 0)
        pltpu.async_copy(src.at[gxs[0]], rows[0], gsems[0])
        wait_gidx(1, 1)
        pltpu.async_copy(src.at[gxs[1]], rows[1], gsems[1])

        def octet(o, carry):
            for b8 in range(8):
                chunk(8 * o + b8, b8)
            return carry

        lax.fori_loop(0, PC // 8, octet, 0)
        for i in range(PC - PC % 8, PC):
            chunk(i, i % 8)
        for i in range(PC - 2, PC):
            pltpu.make_async_copy(rows[i % 4], acc.at[sxs[i % 8]],
                                  ssems[i % 4]).wait()
        plsc.subcore_barrier()

        coff = c * ACC_R

        def write_out(ch):
            r0 = ch * ROWS
            pltpu.sync_copy(acc.at[pl.ds(r0, ROWS)],
                            out.at[pl.ds(coff + r0, ROWS)])

        _strided_chunks(s, ACC_R // ROWS, write_out)

    return pagg


@functools.lru_cache(maxsize=None)
def _make_tc_combine_scale(M, RM):
    nb = M // RM

    def body(p0_ref, p1_ref, de_ref, o_ref):
        d = de_ref[:, 0:1]
        dinv = jnp.where(d > 0.0, 1.0 / d, 0.0)
        o_ref[...] = (p0_ref[...] + p1_ref[...]) * dinv

    return pl.pallas_call(
        body,
        grid=(nb,),
        in_specs=[
            pl.BlockSpec((RM, 128), lambda i: (i, 0)),
            pl.BlockSpec((RM, 128), lambda i: (nb + i, 0)),
            pl.BlockSpec((RM, 128), lambda i: (i, 0)),
        ],
        out_specs=pl.BlockSpec((RM, 128), lambda i: (i, 0)),
        out_shape=jax.ShapeDtypeStruct((M, 128), jnp.float32),
    )


@functools.lru_cache(maxsize=None)
def _make_tc_fin(N, D, R):
    nb = N // R

    def body(yl_ref, yr_ref, dv_ref, w_ref, b_ref, o_ref):
        sc = _scale_from(dv_ref)
        z = jnp.concatenate([yl_ref[...], yr_ref[...]], axis=1) * sc
        z = jnp.maximum(z, 0.0)
        h = jnp.dot(z.astype(jnp.bfloat16),
                    w_ref[...].astype(jnp.bfloat16),
                    preferred_element_type=jnp.float32)
        o_ref[...] = (h + b_ref[0]) * sc

    return pl.pallas_call(
        body,
        grid=(nb,),
        in_specs=[
            pl.BlockSpec((R, 128), lambda i: (i, 0)),
            pl.BlockSpec((R, 128), lambda i: (nb + i, 0)),
            pl.BlockSpec((R, 128), lambda i: (i, 0)),
            pl.BlockSpec((D, 128), lambda i: (0, 0)),
            pl.BlockSpec((1, 1, 128), lambda i: (0, 0, 0)),
        ],
        out_specs=pl.BlockSpec((R, 128), lambda i: (i, 0)),
        out_shape=jax.ShapeDtypeStruct((N, 128), jnp.float32),
    )


@functools.lru_cache(maxsize=None)
def _make_tc_softmax2(N, R, C):
    nb = N // R

    def body(p0_ref, p1_ref, dv_ref, o_ref):
        sc = _scale_from(dv_ref)
        z = (p0_ref[...] + p1_ref[...]) * sc
        lg = z[:, :C]
        m = jnp.max(lg, axis=1, keepdims=True)
        e = jnp.exp(lg - m)
        p = e / jnp.sum(e, axis=1, keepdims=True)
        o_ref[...] = jnp.concatenate(
            [p, jnp.zeros((R, 128 - C), jnp.float32)], axis=1)

    return pl.pallas_call(
        body,
        grid=(nb,),
        in_specs=[
            pl.BlockSpec((R, 128), lambda i: (i, 0)),
            pl.BlockSpec((R, 128), lambda i: (nb + i, 0)),
            pl.BlockSpec((R, 128), lambda i: (i, 0)),
        ],
        out_specs=pl.BlockSpec((R, 128), lambda i: (i, 0)),
        out_shape=jax.ShapeDtypeStruct((N, 128), jnp.float32),
    )


def _scale_from(dv_ref):
    dv = dv_ref[:, 0:1]
    return jnp.where(dv > 0.0, lax.rsqrt(dv), 0.0)


@functools.lru_cache(maxsize=None)
def _make_tc_pre(N, D, R):
    nb = N // R

    def body(x_ref, dv_ref, w_ref, b_ref, o_ref):
        sc = _scale_from(dv_ref)
        h = jnp.dot(x_ref[...].astype(jnp.bfloat16),
                    w_ref[...].astype(jnp.bfloat16),
                    preferred_element_type=jnp.float32)
        o_ref[...] = (h + b_ref[0]) * sc

    return pl.pallas_call(
        body,
        grid=(2, nb),
        in_specs=[
            pl.BlockSpec((R, D), lambda j, i: (i, 0)),
            pl.BlockSpec((R, 128), lambda j, i: (i, 0)),
            pl.BlockSpec((D, 128), lambda j, i: (0, j)),
            pl.BlockSpec((1, 1, 128), lambda j, i: (j, 0, 0)),
        ],
        out_specs=pl.BlockSpec((R, 128), lambda j, i: (j * nb + i, 0)),
        out_shape=jax.ShapeDtypeStruct((2 * N, 128), jnp.float32),
    )


@functools.lru_cache(maxsize=None)
def _make_tc_mid(N, D, R, Wcols):
    nb = N // R

    def body(yl_ref, yr_ref, dv_ref, w_ref, b_ref, o_ref):
        sc = _scale_from(dv_ref)
        z = jnp.concatenate([yl_ref[...], yr_ref[...]], axis=1) * sc
        z = jnp.maximum(z, 0.0)
        h = jnp.dot(z.astype(jnp.bfloat16),
                    w_ref[...].astype(jnp.bfloat16),
                    preferred_element_type=jnp.float32)
        o_ref[...] = (h + b_ref[0]) * sc

    return pl.pallas_call(
        body,
        grid=(2, nb),
        in_specs=[
            pl.BlockSpec((R, 128), lambda j, i: (i, 0)),
            pl.BlockSpec((R, 128), lambda j, i: (nb + i, 0)),
            pl.BlockSpec((R, 128), lambda j, i: (i, 0)),
            pl.BlockSpec((D, Wcols), lambda j, i: (0, j)),
            pl.BlockSpec((1, 1, Wcols), lambda j, i: (j, 0, 0)),
        ],
        out_specs=pl.BlockSpec((R, Wcols), lambda j, i: (j * nb + i, 0)),
        out_shape=jax.ShapeDtypeStruct((2 * N, Wcols), jnp.float32),
    )


@functools.lru_cache(maxsize=None)
def _make_tc_softmax(N, R, C):
    nb = N // R

    def body(yl_ref, dv_ref, o_ref):
        sc = _scale_from(dv_ref)
        z = yl_ref[...] * sc
        lg = z[:, :C]
        m = jnp.max(lg, axis=1, keepdims=True)
        e = jnp.exp(lg - m)
        p = e / jnp.sum(e, axis=1, keepdims=True)
        o_ref[...] = jnp.concatenate(
            [p, jnp.zeros((R, 128 - C), jnp.float32)], axis=1)

    return pl.pallas_call(
        body,
        grid=(nb,),
        in_specs=[
            pl.BlockSpec((R, 128), lambda i: (i, 0)),
            pl.BlockSpec((R, 128), lambda i: (i, 0)),
        ],
        out_specs=pl.BlockSpec((R, 128), lambda i: (i, 0)),
        out_shape=jax.ShapeDtypeStruct((N, 128), jnp.float32),
    )


def kernel(X, v_ids, e_ids, W0, b0, W1, b1, Wf, bf):
    N, D = X.shape
    NNZ = v_ids.shape[0]
    M = M_EDGES
    C = Wf.shape[1]
    R = 400

    ids2 = jnp.concatenate([v_ids, e_ids])
    dv16, de16 = _make_degree_kernel(N, M, NNZ)(ids2)

    vg1 = jnp.concatenate([v_ids, v_ids + N])
    eg1 = jnp.concatenate([e_ids, e_ids + M])

    de16s = de16[:, :16]
    smooth_d = _make_smooth_kernel(N, M, NNZ, D // 2)

    xh2 = _make_tc_pre(N, D, R)(X, dv16, W0, b0.reshape(2, 1, 128))
    y2, _ = smooth_d(xh2, vg1, e_ids, eg1, v_ids, de16s)

    xh2 = _make_tc_mid(N, D, R, 128)(y2, y2, dv16, W1,
                                     b1.reshape(2, 1, 128))
    y2, _ = smooth_d(xh2, vg1, e_ids, eg1, v_ids, de16s)

    wfp = jnp.pad(Wf, ((0, 0), (0, 128 - C)))
    bfp = jnp.pad(bf, (0, 128 - C)).reshape(1, 1, 128)
    xf = _make_tc_fin(N, D, R)(y2, y2, dv16, wfp, bfp)
    pe = _make_partial_agg(NNZ, M)(xf, v_ids, e_ids)
    es = _make_tc_combine_scale(M, 1000)(pe, pe, de16)
    py = _make_partial_agg(NNZ, N)(es, e_ids, v_ids)
    out = _make_tc_softmax2(N, R, C)(py, py, dv16)
    return out[:, :C]


# speedup vs baseline: 1.0010x; 1.0010x over previous
import functools

import jax
import jax.numpy as jnp
from jax import lax
from jax.experimental import pallas as pl
from jax.experimental.pallas import tpu as pltpu
from jax.experimental.pallas import tpu_sc as plsc

M_EDGES = 5000
NS = 16
K = 80
ROWS = 8


def _strided_chunks(sid, total_chunks, body):
    per = (total_chunks + NS - 1) // NS

    def loop_body(i, carry):
        ch = sid + i * NS

        @pl.when(ch < total_chunks)
        def _():
            body(ch)

        return carry

    lax.fori_loop(0, per, loop_body, 0)


@functools.lru_cache(maxsize=None)
def _make_degree_kernel(N, M, NNZ):
    PT = NNZ // NS
    PC = PT // K
    assert NNZ % NS == 0 and PT % K == 0 and PC % 8 == 5
    mesh = plsc.VectorSubcoreMesh(core_axis_name="c", subcore_axis_name="s")

    @functools.partial(
        pl.kernel,
        out_type=(jax.ShapeDtypeStruct((N, 128), jnp.float32),
                  jax.ShapeDtypeStruct((M, 128), jnp.float32)),
        mesh=mesh,
        scratch_types=(
            [pltpu.VMEM((K,), jnp.int32)] * 8
            + [pltpu.VMEM((ROWS, 128), jnp.float32),
               pltpu.VMEM((K, 128), jnp.float32),
               pltpu.VMEM_SHARED((N, 128), jnp.float32)]
            + [pltpu.SemaphoreType.DMA] * 12
        ),
    )
    def deg_kernel(ids2, dv16, de16,
                   x0, x1, x2, x3, x4, x5, x6, x7, zb, ones, hist,
                   s0, s1, s2, s3, j0, j1, j2, j3, j4, j5, j6, j7):
        c = lax.axis_index("c")
        s = lax.axis_index("s")
        sxs = (x0, x1, x2, x3, x4, x5, x6, x7)
        ssems = (s0, s1, s2, s3)
        jsems = (j0, j1, j2, j3, j4, j5, j6, j7)
        base = c * NNZ + s * PT
        for r in range(ROWS):
            for cc in range(8):
                zb[r, pl.ds(cc * 16, 16)] = jnp.zeros((16,), jnp.float32)
        for r in range(K):
            for cc in range(8):
                ones[r, pl.ds(cc * 16, 16)] = jnp.ones((16,), jnp.float32)

        def zero_chunk(ch):
            pltpu.sync_copy(zb, hist.at[pl.ds(ch * ROWS, ROWS)])

        _strided_chunks(s, N // ROWS, zero_chunk)
        plsc.subcore_barrier()

        def load_idx(i, b8):
            pltpu.async_copy(ids2.at[pl.ds(base + i * K, K)], sxs[b8],
                             jsems[b8])

        def wait_idx(i, b8):
            pltpu.make_async_copy(ids2.at[pl.ds(base + i * K, K)],
                                  sxs[b8], jsems[b8]).wait()

        def chunk(i, b8):
            b4 = b8 % 4
            nb = (b8 + 4) % 8

            @pl.when(i >= 4)
            def _():
                pltpu.make_async_copy(ones, hist.at[sxs[nb]],
                                      ssems[b4]).wait()

            @pl.when(i + 4 < PC)
            def _():
                load_idx(i + 4, nb)

            wait_idx(i, b8)
            pltpu.async_copy(ones, hist.at[sxs[b8]], ssems[b4], add=True)

        for b in range(4):
            load_idx(b, b)

        def octet(o, carry):
            for b8 in range(8):
                chunk(8 * o + b8, b8)
            return carry

        lax.fori_loop(0, PC // 8, octet, 0)
        for i in range(PC - PC % 8, PC):
            chunk(i, i % 8)
        for i in range(PC - 4, PC):
            pltpu.make_async_copy(ones, hist.at[sxs[i % 8]],
                                  ssems[i % 4]).wait()
        plsc.subcore_barrier()

        @pl.when(c == 0)
        def _():
            def out_v(ch):
                pltpu.sync_copy(hist.at[pl.ds(ch * ROWS, ROWS)],
                                dv16.at[pl.ds(ch * ROWS, ROWS)])

            _strided_chunks(s, N // ROWS, out_v)

        @pl.when(c == 1)
        def _():
            def out_e(ch):
                pltpu.sync_copy(hist.at[pl.ds(ch * ROWS, ROWS)],
                                de16.at[pl.ds(ch * ROWS, ROWS)])

            _strided_chunks(s, M // ROWS, out_e)

    return deg_kernel


@functools.lru_cache(maxsize=None)
def _make_smooth_kernel(N, M, NNZ, Dh):
    PT = NNZ // NS
    PC = PT // K
    assert NNZ % NS == 0 and PT % K == 0 and PC % 8 == 5
    assert Dh % 16 == 0
    mesh = plsc.VectorSubcoreMesh(core_axis_name="c", subcore_axis_name="s")

    @functools.partial(
        pl.kernel,
        out_type=(jax.ShapeDtypeStruct((2 * N, Dh), jnp.float32),
                  jax.ShapeDtypeStruct((2 * M, Dh), jnp.float32)),
        mesh=mesh,
        scratch_types=(
            [pltpu.VMEM((K,), jnp.int32)] * 16
            + [pltpu.VMEM((K, Dh), jnp.float32)] * 4
            + [pltpu.VMEM((ROWS, Dh), jnp.float32),
               pltpu.VMEM((ROWS, 16), jnp.float32),
               pltpu.VMEM_SHARED((N, Dh), jnp.float32)]
            + [pltpu.SemaphoreType.DMA] * 24
        ),
    )
    def smooth(xh2, vg1, es1, eg1, vs1, de16, y2, eagg2, *bufs):
        gxs = bufs[0:8]
        sxs = bufs[8:16]
        rows = bufs[16:20]
        ebuf, dbuf, acc = bufs[20:23]
        gsems = bufs[23:27]
        ssems = bufs[27:31]
        isems = bufs[31:39]
        jsems = bufs[39:47]
        c = lax.axis_index("c")
        s = lax.axis_index("s")
        gbase = (c * NS + s) * PT
        sbase = s * PT

        for r in range(ROWS):
            for cc in range(Dh // 16):
                ebuf[r, pl.ds(cc * 16, 16)] = jnp.zeros((16,), jnp.float32)

        def zero_acc(ch):
            pltpu.sync_copy(ebuf, acc.at[pl.ds(ch * ROWS, ROWS)])

        _strided_chunks(s, M // ROWS, zero_acc)
        plsc.subcore_barrier()

        def pipe_phase(src, g1, s1):
            def load_gidx(i, b):
                pltpu.async_copy(g1.at[pl.ds(gbase + i * K, K)], gxs[b],
                                 isems[b])

            def wait_gidx(i, b):
                pltpu.make_async_copy(g1.at[pl.ds(gbase + i * K, K)],
                                      gxs[b], isems[b]).wait()

            def load_sidx(i, b):
                pltpu.async_copy(s1.at[pl.ds(sbase + i * K, K)], sxs[b],
                                 jsems[b])

            def wait_sidx(i, b):
                pltpu.make_async_copy(s1.at[pl.ds(sbase + i * K, K)],
                                      sxs[b], jsems[b]).wait()

            def chunk(i, b8):
                b4 = b8 % 4
                g2 = (b4 + 2) % 4

                @pl.when(i >= 2)
                def _():
                    pltpu.make_async_copy(rows[g2],
                                          acc.at[sxs[(b8 + 6) % 8]],
                                          ssems[g2]).wait()

                @pl.when(i + 6 < PC)
                def _():
                    load_sidx(i + 6, (b8 + 6) % 8)

                @pl.when(i + 2 < PC)
                def _():
                    wait_gidx(i + 2, (b8 + 2) % 8)
                    pltpu.async_copy(src.at[gxs[(b8 + 2) % 8]], rows[g2],
                                     gsems[g2])

                pltpu.make_async_copy(src.at[gxs[b8]], rows[b4],
                                      gsems[b4]).wait()

                @pl.when(i + 8 < PC)
                def _():
                    load_gidx(i + 8, b8)

                wait_sidx(i, b8)
                pltpu.async_copy(rows[b4], acc.at[sxs[b8]], ssems[b4],
                                 add=True)

            for b in range(8):
                load_gidx(b, b)
                load_sidx(b, b)
            wait_gidx(0, 0)
            pltpu.async_copy(src.at[gxs[0]], rows[0], gsems[0])
            wait_gidx(1, 1)
            pltpu.async_copy(src.at[gxs[1]], rows[1], gsems[1])

            def octet(o, carry):
                for b8 in range(8):
                    chunk(8 * o + b8, b8)
                return carry

            lax.fori_loop(0, PC // 8, octet, 0)
            for i in range(PC - PC % 8, PC):
                chunk(i, i % 8)
            for i in range(PC - 2, PC):
                pltpu.make_async_copy(rows[i % 4], acc.at[sxs[i % 8]],
                                      ssems[i % 4]).wait()

        eoff = c * M
        pipe_phase(xh2, vg1, es1)
        plsc.subcore_barrier()

        def scale_chunk(ch):
            r0 = ch * ROWS
            pltpu.sync_copy(acc.at[pl.ds(r0, ROWS)], ebuf)
            pltpu.sync_copy(de16.at[pl.ds(r0, ROWS)], dbuf)
            for r in range(ROWS):
                d = dbuf[r, pl.ds(0, 16)]
                dinv = jnp.where(d > 0.0, 1.0 / d, 0.0)
                for cc in range(Dh // 16):
                    ebuf[r, pl.ds(cc * 16, 16)] = (
                        ebuf[r, pl.ds(cc * 16, 16)] * dinv)
            pltpu.sync_copy(ebuf, eagg2.at[pl.ds(eoff + r0, ROWS)])

        _strided_chunks(s, M // ROWS, scale_chunk)
        plsc.subcore_barrier()

        for r in range(ROWS):
            for cc in range(Dh // 16):
                ebuf[r, pl.ds(cc * 16, 16)] = jnp.zeros((16,), jnp.float32)

        def zero_acc2(ch):
            pltpu.sync_copy(ebuf, acc.at[pl.ds(ch * ROWS, ROWS)])

        _strided_chunks(s, N // ROWS, zero_acc2)
        plsc.subcore_barrier()

        pipe_phase(eagg2, eg1, vs1)
        plsc.subcore_barrier()

        coff = c * N

        def write_out(ch):
            r0 = ch * ROWS
            pltpu.sync_copy(acc.at[pl.ds(r0, ROWS)],
                            y2.at[pl.ds(coff + r0, ROWS)])

        _strided_chunks(s, N // ROWS, write_out)

    return smooth


@functools.lru_cache(maxsize=None)
def _make_partial_agg(NNZ, ACC_R):
    K2 = 40
    PT2 = NNZ // (2 * NS)
    PC = PT2 // K2
    assert NNZ % (2 * NS) == 0 and PT2 % K2 == 0 and PC % 8 == 5
    mesh = plsc.VectorSubcoreMesh(core_axis_name="c", subcore_axis_name="s")

    @functools.partial(
        pl.kernel,
        out_type=jax.ShapeDtypeStruct((2 * ACC_R, 128), jnp.float32),
        mesh=mesh,
        scratch_types=(
            [pltpu.VMEM((K2,), jnp.int32)] * 16
            + [pltpu.VMEM((K2, 128), jnp.float32)] * 4
            + [pltpu.VMEM((ROWS, 128), jnp.float32),
               pltpu.VMEM_SHARED((ACC_R, 128), jnp.float32)]
            + [pltpu.SemaphoreType.DMA] * 24
        ),
    )
    def pagg(src, g1, s1, out, *bufs):
        gxs = bufs[0:8]
        sxs = bufs[8:16]
        rows = bufs[16:20]
        zbuf, acc = bufs[20:22]
        gsems = bufs[22:26]
        ssems = bufs[26:30]
        isems = bufs[30:38]
        jsems = bufs[38:46]
        c = lax.axis_index("c")
        s = lax.axis_index("s")
        base = (c * NS + s) * PT2

        for r in range(ROWS):
            for cc in range(8):
                zbuf[r, pl.ds(cc * 16, 16)] = jnp.zeros((16,), jnp.float32)

        def zero_acc(ch):
            pltpu.sync_copy(zbuf, acc.at[pl.ds(ch * ROWS, ROWS)])

        _strided_chunks(s, ACC_R // ROWS, zero_acc)
        plsc.subcore_barrier()

        def load_gidx(i, b):
            pltpu.async_copy(g1.at[pl.ds(base + i * K2, K2)], gxs[b],
                             isems[b])

        def wait_gidx(i, b):
            pltpu.make_async_copy(g1.at[pl.ds(base + i * K2, K2)],
                                  gxs[b], isems[b]).wait()

        def load_sidx(i, b):
            pltpu.async_copy(s1.at[pl.ds(base + i * K2, K2)], sxs[b],
                             jsems[b])

        def wait_sidx(i, b):
            pltpu.make_async_copy(s1.at[pl.ds(base + i * K2, K2)],
                                  sxs[b], jsems[b]).wait()

        def chunk(i, b8):
            b4 = b8 % 4
            g2 = (b4 + 2) % 4

            @pl.when(i >= 2)
            def _():
                pltpu.make_async_copy(rows[g2],
                                      acc.at[sxs[(b8 + 6) % 8]],
                                      ssems[g2]).wait()

            @pl.when(i + 6 < PC)
            def _():
                load_sidx(i + 6, (b8 + 6) % 8)

            @pl.when(i + 2 < PC)
            def _():
                wait_gidx(i + 2, (b8 + 2) % 8)
                pltpu.async_copy(src.at[gxs[(b8 + 2) % 8]], rows[g2],
                                 gsems[g2])

            pltpu.make_async_copy(src.at[gxs[b8]], rows[b4],
                                  gsems[b4]).wait()

            @pl.when(i + 8 < PC)
            def _():
                load_gidx(i + 8, b8)

            wait_sidx(i, b8)
            pltpu.async_copy(rows[b4], acc.at[sxs[b8]], ssems[b4],
                             add=True)

        for b in range(8):
            load_gidx(b, b)
            load_sidx(b, b)
        wait_gidx(0, 0)
        pltpu.async_copy(src.at[gxs[0]], rows[0], gsems[0])
        wait_gidx(1, 1)
        pltpu.async_copy(src.at[gxs[1]], rows[1], gsems[1])

        def octet(o, carry):
            for b8 in range(8):
                chunk(8 * o + b8, b8)
            return carry

        lax.fori_loop(0, PC // 8, octet, 0)
        for i in range(PC - PC % 8, PC):
            chunk(i, i % 8)
        for i in range(PC - 2, PC):
            pltpu.make_async_copy(rows[i % 4], acc.at[sxs[i % 8]],
                                  ssems[i % 4]).wait()
        plsc.subcore_barrier()

        coff = c * ACC_R

        def write_out(ch):
            r0 = ch * ROWS
            pltpu.sync_copy(acc.at[pl.ds(r0, ROWS)],
                            out.at[pl.ds(coff + r0, ROWS)])

        _strided_chunks(s, ACC_R // ROWS, write_out)

    return pagg


@functools.lru_cache(maxsize=None)
def _make_tc_combine_scale(M, RM):
    nb = M // RM

    def body(p0_ref, p1_ref, de_ref, o_ref):
        d = de_ref[:, 0:1]
        dinv = jnp.where(d > 0.0, 1.0 / d, 0.0)
        o_ref[...] = (p0_ref[...] + p1_ref[...]) * dinv

    return pl.pallas_call(
        body,
        grid=(nb,),
        in_specs=[
            pl.BlockSpec((RM, 128), lambda i: (i, 0)),
            pl.BlockSpec((RM, 128), lambda i: (nb + i, 0)),
            pl.BlockSpec((RM, 128), lambda i: (i, 0)),
        ],
        out_specs=pl.BlockSpec((RM, 128), lambda i: (i, 0)),
        out_shape=jax.ShapeDtypeStruct((M, 128), jnp.float32),
    )


@functools.lru_cache(maxsize=None)
def _make_tc_fin(N, D, R):
    nb = N // R

    def body(yl_ref, yr_ref, dv_ref, w_ref, b_ref, o_ref):
        sc = _scale_from(dv_ref)
        z = jnp.concatenate([yl_ref[...], yr_ref[...]], axis=1) * sc
        z = jnp.maximum(z, 0.0)
        h = jnp.dot(z, w_ref[...], preferred_element_type=jnp.float32)
        o_ref[...] = (h + b_ref[0]) * sc

    return pl.pallas_call(
        body,
        grid=(nb,),
        in_specs=[
            pl.BlockSpec((R, 128), lambda i: (i, 0)),
            pl.BlockSpec((R, 128), lambda i: (nb + i, 0)),
            pl.BlockSpec((R, 128), lambda i: (i, 0)),
            pl.BlockSpec((D, 128), lambda i: (0, 0)),
            pl.BlockSpec((1, 1, 128), lambda i: (0, 0, 0)),
        ],
        out_specs=pl.BlockSpec((R, 128), lambda i: (i, 0)),
        out_shape=jax.ShapeDtypeStruct((N, 128), jnp.float32),
    )


@functools.lru_cache(maxsize=None)
def _make_tc_softmax2(N, R, C):
    nb = N // R

    def body(p0_ref, p1_ref, dv_ref, o_ref):
        sc = _scale_from(dv_ref)
        z = (p0_ref[...] + p1_ref[...]) * sc
        lg = z[:, :C]
        m = jnp.max(lg, axis=1, keepdims=True)
        e = jnp.exp(lg - m)
        p = e / jnp.sum(e, axis=1, keepdims=True)
        o_ref[...] = jnp.concatenate(
            [p, jnp.zeros((R, 128 - C), jnp.float32)], axis=1)

    return pl.pallas_call(
        body,
        grid=(nb,),
        in_specs=[
            pl.BlockSpec((R, 128), lambda i: (i, 0)),
            pl.BlockSpec((R, 128), lambda i: (nb + i, 0)),
            pl.BlockSpec((R, 128), lambda i: (i, 0)),
        ],
        out_specs=pl.BlockSpec((R, 128), lambda i: (i, 0)),
        out_shape=jax.ShapeDtypeStruct((N, 128), jnp.float32),
    )


def _scale_from(dv_ref):
    dv = dv_ref[:, 0:1]
    return jnp.where(dv > 0.0, lax.rsqrt(dv), 0.0)


@functools.lru_cache(maxsize=None)
def _make_tc_pre(N, D, R):
    nb = N // R

    def body(x_ref, dv_ref, w_ref, b_ref, o_ref):
        sc = _scale_from(dv_ref)
        h = jnp.dot(x_ref[...], w_ref[...],
                    preferred_element_type=jnp.float32)
        o_ref[...] = (h + b_ref[0]) * sc

    return pl.pallas_call(
        body,
        grid=(2, nb),
        in_specs=[
            pl.BlockSpec((R, D), lambda j, i: (i, 0)),
            pl.BlockSpec((R, 128), lambda j, i: (i, 0)),
            pl.BlockSpec((D, 128), lambda j, i: (0, j)),
            pl.BlockSpec((1, 1, 128), lambda j, i: (j, 0, 0)),
        ],
        out_specs=pl.BlockSpec((R, 128), lambda j, i: (j * nb + i, 0)),
        out_shape=jax.ShapeDtypeStruct((2 * N, 128), jnp.float32),
    )


@functools.lru_cache(maxsize=None)
def _make_tc_mid(N, D, R, Wcols):
    nb = N // R

    def body(yl_ref, yr_ref, dv_ref, w_ref, b_ref, o_ref):
        sc = _scale_from(dv_ref)
        z = jnp.concatenate([yl_ref[...], yr_ref[...]], axis=1) * sc
        z = jnp.maximum(z, 0.0)
        h = jnp.dot(z, w_ref[...], preferred_element_type=jnp.float32)
        o_ref[...] = (h + b_ref[0]) * sc

    return pl.pallas_call(
        body,
        grid=(2, nb),
        in_specs=[
            pl.BlockSpec((R, 128), lambda j, i: (i, 0)),
            pl.BlockSpec((R, 128), lambda j, i: (nb + i, 0)),
            pl.BlockSpec((R, 128), lambda j, i: (i, 0)),
            pl.BlockSpec((D, Wcols), lambda j, i: (0, j)),
            pl.BlockSpec((1, 1, Wcols), lambda j, i: (j, 0, 0)),
        ],
        out_specs=pl.BlockSpec((R, Wcols), lambda j, i: (j * nb + i, 0)),
        out_shape=jax.ShapeDtypeStruct((2 * N, Wcols), jnp.float32),
    )


@functools.lru_cache(maxsize=None)
def _make_tc_softmax(N, R, C):
    nb = N // R

    def body(yl_ref, dv_ref, o_ref):
        sc = _scale_from(dv_ref)
        z = yl_ref[...] * sc
        lg = z[:, :C]
        m = jnp.max(lg, axis=1, keepdims=True)
        e = jnp.exp(lg - m)
        p = e / jnp.sum(e, axis=1, keepdims=True)
        o_ref[...] = jnp.concatenate(
            [p, jnp.zeros((R, 128 - C), jnp.float32)], axis=1)

    return pl.pallas_call(
        body,
        grid=(nb,),
        in_specs=[
            pl.BlockSpec((R, 128), lambda i: (i, 0)),
            pl.BlockSpec((R, 128), lambda i: (i, 0)),
        ],
        out_specs=pl.BlockSpec((R, 128), lambda i: (i, 0)),
        out_shape=jax.ShapeDtypeStruct((N, 128), jnp.float32),
    )


def kernel(X, v_ids, e_ids, W0, b0, W1, b1, Wf, bf):
    N, D = X.shape
    NNZ = v_ids.shape[0]
    M = M_EDGES
    C = Wf.shape[1]
    R = 400

    ids2 = jnp.concatenate([v_ids, e_ids])
    dv16, de16 = _make_degree_kernel(N, M, NNZ)(ids2)

    vg1 = jnp.concatenate([v_ids, v_ids + N])
    eg1 = jnp.concatenate([e_ids, e_ids + M])

    de16s = de16[:, :16]
    smooth_d = _make_smooth_kernel(N, M, NNZ, D // 2)

    xh2 = _make_tc_pre(N, D, R)(X, dv16, W0, b0.reshape(2, 1, 128))
    y2, _ = smooth_d(xh2, vg1, e_ids, eg1, v_ids, de16s)

    xh2 = _make_tc_mid(N, D, R, 128)(y2, y2, dv16, W1,
                                     b1.reshape(2, 1, 128))
    y2, _ = smooth_d(xh2, vg1, e_ids, eg1, v_ids, de16s)

    wfp = jnp.pad(Wf, ((0, 0), (0, 128 - C)))
    bfp = jnp.pad(bf, (0, 128 - C)).reshape(1, 1, 128)
    xf = _make_tc_fin(N, D, R)(y2, y2, dv16, wfp, bfp)
    pe = _make_partial_agg(NNZ, M)(xf, v_ids, e_ids)
    es = _make_tc_combine_scale(M, 1000)(pe, pe, de16)
    py = _make_partial_agg(NNZ, N)(es, e_ids, v_ids)
    out = _make_tc_softmax2(N, R, C)(py, py, dv16)
    return out[:, :C]
